# Initial kernel scaffold; baseline (speedup 1.0000x reference)
import jax
import jax.numpy as jnp
from jax.experimental import pallas as pl


def kernel(phrase, phrase_emb_weight, pos_emb_weight):
    raise NotImplementedError("write your pallas kernel here")


# speedup vs baseline: 5.4302x; 5.4302x over previous
import functools

import jax
import jax.numpy as jnp
from jax import lax
from jax.experimental import pallas as pl
from jax.experimental.pallas import tpu as pltpu
from jax.experimental.pallas import tpu_sc as plsc

_D = 64
_B = 4096
_L = 50
_NW = 32
_ROWS = _B * _L
_RPW = _ROWS // _NW
_KG = 100
_K = 200
_J = _RPW // _K
_JG = _RPW // _KG


def _make_sc_embed():
  mesh = plsc.VectorSubcoreMesh(core_axis_name="c", subcore_axis_name="s")

  @functools.partial(
      pl.kernel,
      mesh=mesh,
      compiler_params=pltpu.CompilerParams(use_tc_tiling_on_sc=False),
      out_type=jax.ShapeDtypeStruct((_ROWS, _D), jnp.float32),
      scratch_types=[
          pltpu.VMEM((_JG, _KG), jnp.int32),
          pltpu.VMEM((_L, _D), jnp.float32),
          pltpu.VMEM((_K, _D), jnp.float32),
          pltpu.SemaphoreType.DMA,
      ],
  )
  def embed(idx_hbm, table_hbm, pos_hbm, out_hbm, idx_v, pos_v, rows_v, gsem):
    cid = lax.axis_index("c")
    sid = lax.axis_index("s")
    wid = sid * 2 + cid
    base = wid * _RPW
    pltpu.sync_copy(idx_hbm.at[wid], idx_v)
    pltpu.sync_copy(pos_hbm, pos_v)

    def chunk(j, carry):
      c0 = pltpu.async_copy(
          table_hbm.at[idx_v.at[2 * j]], rows_v.at[pl.ds(0, _KG)], gsem)
      c1 = pltpu.async_copy(
          table_hbm.at[idx_v.at[2 * j + 1]], rows_v.at[pl.ds(_KG, _KG)], gsem)
      c0.wait()
      c1.wait()
      for r in range(_K):
        lp = r % _L
        for s in range(_D // 16):
          sl = pl.ds(16 * s, 16)
          rows_v[r, sl] = rows_v[r, sl] + pos_v[lp, sl]
      pltpu.sync_copy(rows_v, out_hbm.at[pl.ds(base + j * _K, _K)])
      return carry

    lax.fori_loop(0, _J, chunk, 0)

  return embed


_sc_embed = _make_sc_embed()


def kernel(phrase, phrase_emb_weight, pos_emb_weight):
  idx = phrase.astype(jnp.int32).reshape(_NW, _JG, _KG)
  out = _sc_embed(idx, phrase_emb_weight, pos_emb_weight)
  return out.reshape(_B, _L, _D)


# speedup vs baseline: 6.4817x; 1.1936x over previous
import functools

import jax
import jax.numpy as jnp
from jax import lax
from jax.experimental import pallas as pl
from jax.experimental.pallas import tpu as pltpu
from jax.experimental.pallas import tpu_sc as plsc

_D = 64
_B = 4096
_L = 50
_NW = 32
_PPW = _B // _NW
_CP = 2
_J = _PPW // _CP
_NBUF = 2
_G = _J // _NBUF


def _make_sc_embed():
  mesh = plsc.VectorSubcoreMesh(core_axis_name="c", subcore_axis_name="s")

  @functools.partial(
      pl.kernel,
      mesh=mesh,
      compiler_params=pltpu.CompilerParams(use_tc_tiling_on_sc=False),
      out_type=jax.ShapeDtypeStruct((_B, _L, _D), jnp.float32),
      scratch_types=[
          pltpu.VMEM((_PPW, _L), jnp.int32),
          pltpu.VMEM((_L, _D), jnp.float32),
          pltpu.VMEM((_NBUF, _CP, _L, _D), jnp.float32),
          pltpu.VMEM((_NBUF, _CP, _L, _D), jnp.float32),
          pltpu.SemaphoreType.DMA,
          pltpu.SemaphoreType.DMA,
          pltpu.SemaphoreType.DMA,
          pltpu.SemaphoreType.DMA,
      ],
  )
  def embed(idx_hbm, table_hbm, pos_hbm, out_hbm, idx_v, pos_v, gbuf, obuf,
            gsem0, gsem1, ssem0, ssem1):
    gsems = (gsem0, gsem1)
    ssems = (ssem0, ssem1)
    cid = lax.axis_index("c")
    sid = lax.axis_index("s")
    wid = sid * 2 + cid
    pbase = wid * _PPW
    pltpu.sync_copy(idx_hbm.at[pl.ds(pbase, _PPW)], idx_v)
    pltpu.sync_copy(pos_hbm, pos_v)

    def gather_desc(local_p, slot, q, sem):
      return pltpu.make_async_copy(
          table_hbm.at[idx_v.at[local_p]], gbuf.at[slot, q], sem)

    def store_desc(slot, local_p, sem):
      return pltpu.make_async_copy(
          obuf.at[slot], out_hbm.at[pl.ds(pbase + local_p, _CP)], sem)

    for b in range(_NBUF):
      for q in range(_CP):
        gather_desc(b * _CP + q, b, q, gsems[b]).start()

    def body(g, carry):
      for b in range(_NBUF):
        lp = (g * _NBUF + b) * _CP
        for q in range(_CP):
          gather_desc(lp + q, b, q, gsems[b]).wait()

        @pl.when(g > 0)
        def _():
          store_desc(b, lp - _NBUF * _CP, ssems[b]).wait()

        for q in range(_CP):
          for r in range(_L):
            for s in range(_D // 16):
              sl = pl.ds(16 * s, 16)
              obuf[b, q, r, sl] = gbuf[b, q, r, sl] + pos_v[r, sl]

        @pl.when(g < _G - 1)
        def _():
          nxt = lp + _NBUF * _CP
          for q in range(_CP):
            gather_desc(nxt + q, b, q, gsems[b]).start()

        store_desc(b, lp, ssems[b]).start()
      return carry

    lax.fori_loop(0, _G, body, 0)

    for b in range(_NBUF):
      lp = ((_G - 1) * _NBUF + b) * _CP
      store_desc(b, lp, ssems[b]).wait()

  return embed


_sc_embed = _make_sc_embed()


def kernel(phrase, phrase_emb_weight, pos_emb_weight):
  return _sc_embed(phrase.astype(jnp.int32), phrase_emb_weight,
                   pos_emb_weight)


# speedup vs baseline: 7.5440x; 1.1639x over previous
import functools

import jax
import jax.numpy as jnp
from jax import lax
from jax.experimental import pallas as pl
from jax.experimental.pallas import tpu as pltpu
from jax.experimental.pallas import tpu_sc as plsc

_D = 64
_DP = 128
_B = 4096
_L = 50
_NW = 32
_PPW = _B // _NW
_CP = 2
_J = _PPW // _CP
_NBUF = 2
_G = _J // _NBUF


def _make_sc_embed():
  mesh = plsc.VectorSubcoreMesh(core_axis_name="c", subcore_axis_name="s")

  @functools.partial(
      pl.kernel,
      mesh=mesh,
      compiler_params=pltpu.CompilerParams(use_tc_tiling_on_sc=True),
      out_type=jax.ShapeDtypeStruct((_B, _L, _D), jnp.float32),
      scratch_types=[
          pltpu.VMEM((_PPW, _L), jnp.int32),
          pltpu.VMEM((_L, _D), jnp.float32),
          pltpu.VMEM((_NBUF, _CP, _L, _DP), jnp.float32),
          pltpu.VMEM((_NBUF, _CP, _L, _D), jnp.float32),
          pltpu.SemaphoreType.DMA,
          pltpu.SemaphoreType.DMA,
          pltpu.SemaphoreType.DMA,
          pltpu.SemaphoreType.DMA,
      ],
  )
  def embed(idx_hbm, table_hbm, pos_hbm, out_hbm, idx_v, pos_v, gbuf, obuf,
            gsem0, gsem1, ssem0, ssem1):
    gsems = (gsem0, gsem1)
    ssems = (ssem0, ssem1)
    cid = lax.axis_index("c")
    sid = lax.axis_index("s")
    wid = sid * 2 + cid
    pbase = wid * _PPW
    pltpu.sync_copy(idx_hbm.at[pl.ds(pbase, _PPW)], idx_v)
    pltpu.sync_copy(pos_hbm, pos_v)

    def gather_desc(local_p, slot, q, sem):
      return pltpu.make_async_copy(
          table_hbm.at[idx_v.at[local_p]], gbuf.at[slot, q], sem)

    def store_desc(slot, local_p, sem):
      return pltpu.make_async_copy(
          obuf.at[slot], out_hbm.at[pl.ds(pbase + local_p, _CP)], sem)

    for b in range(_NBUF):
      for q in range(_CP):
        gather_desc(b * _CP + q, b, q, gsems[b]).start()

    def body(g, carry):
      for b in range(_NBUF):
        lp = (g * _NBUF + b) * _CP
        for q in range(_CP):
          gather_desc(lp + q, b, q, gsems[b]).wait()

        @pl.when(g > 0)
        def _():
          store_desc(b, lp - _NBUF * _CP, ssems[b]).wait()

        for q in range(_CP):
          for r in range(_L):
            for s in range(_D // 16):
              sl = pl.ds(16 * s, 16)
              obuf[b, q, r, sl] = gbuf[b, q, r, sl] + pos_v[r, sl]

        @pl.when(g < _G - 1)
        def _():
          nxt = lp + _NBUF * _CP
          for q in range(_CP):
            gather_desc(nxt + q, b, q, gsems[b]).start()

        store_desc(b, lp, ssems[b]).start()
      return carry

    lax.fori_loop(0, _G, body, 0)

    for b in range(_NBUF):
      lp = ((_G - 1) * _NBUF + b) * _CP
      store_desc(b, lp, ssems[b]).wait()

  return embed


_sc_embed = _make_sc_embed()


def kernel(phrase, phrase_emb_weight, pos_emb_weight):
  table_p = jnp.pad(phrase_emb_weight, ((0, 0), (0, _DP - _D)))
  return _sc_embed(phrase.astype(jnp.int32), table_p, pos_emb_weight)


# speedup vs baseline: 8.9298x; 1.1837x over previous
import functools

import jax
import jax.numpy as jnp
from jax import lax
from jax.experimental import pallas as pl
from jax.experimental.pallas import tpu as pltpu
from jax.experimental.pallas import tpu_sc as plsc

_D = 64
_DP = 128
_B = 4096
_L = 50
_NW = 32
_PPW = _B // _NW
_CP = 2
_J = _PPW // _CP
_NBUF = 2
_G = _J // _NBUF


def _make_sc_embed():
  mesh = plsc.VectorSubcoreMesh(core_axis_name="c", subcore_axis_name="s")

  @functools.partial(
      pl.kernel,
      mesh=mesh,
      compiler_params=pltpu.CompilerParams(use_tc_tiling_on_sc=True),
      out_type=jax.ShapeDtypeStruct((_B, _L, _D), jnp.float32),
      scratch_types=[
          pltpu.VMEM((_PPW, _L), jnp.int32),
          pltpu.VMEM((_L, _D), jnp.float32),
          pltpu.VMEM((_NBUF, _CP, _L, _DP), jnp.float32),
          pltpu.VMEM((_NBUF, _CP, _L, _D), jnp.float32),
          pltpu.SemaphoreType.DMA,
          pltpu.SemaphoreType.DMA,
          pltpu.SemaphoreType.DMA,
          pltpu.SemaphoreType.DMA,
      ],
  )
  def embed(idx_hbm, table_hbm, pos_hbm, out_hbm, idx_v, pos_v, gbuf, obuf,
            gsem0, gsem1, ssem0, ssem1):
    gsems = (gsem0, gsem1)
    ssems = (ssem0, ssem1)
    cid = lax.axis_index("c")
    sid = lax.axis_index("s")
    wid = sid * 2 + cid
    pbase = wid * _PPW
    pltpu.sync_copy(idx_hbm.at[pl.ds(pbase, _PPW)], idx_v)
    pltpu.sync_copy(pos_hbm, pos_v)

    def gather_desc(local_p, slot, q, sem):
      return pltpu.make_async_copy(
          table_hbm.at[idx_v.at[local_p]], gbuf.at[slot, q], sem)

    def store_desc(slot, local_p, sem):
      return pltpu.make_async_copy(
          obuf.at[slot], out_hbm.at[pl.ds(pbase + local_p, _CP)], sem)

    for b in range(_NBUF):
      for q in range(_CP):
        gather_desc(b * _CP + q, b, q, gsems[b]).start()

    def body(g, carry):
      for b in range(_NBUF):
        lp = (g * _NBUF + b) * _CP
        for q in range(_CP):
          gather_desc(lp + q, b, q, gsems[b]).wait()

        @pl.when(g > 0)
        def _():
          store_desc(b, lp - _NBUF * _CP, ssems[b]).wait()

        for r in range(_L):
          for s in range(_D // 16):
            sl = pl.ds(16 * s, 16)
            pv = pos_v[r, sl]
            for q in range(_CP):
              obuf[b, q, r, sl] = gbuf[b, q, r, sl] + pv

        @pl.when(g < _G - 1)
        def _():
          nxt = lp + _NBUF * _CP
          for q in range(_CP):
            gather_desc(nxt + q, b, q, gsems[b]).start()

        store_desc(b, lp, ssems[b]).start()
      return carry

    lax.fori_loop(0, _G, body, 0)

    for b in range(_NBUF):
      lp = ((_G - 1) * _NBUF + b) * _CP
      store_desc(b, lp, ssems[b]).wait()

  return embed


_sc_embed = _make_sc_embed()


def kernel(phrase, phrase_emb_weight, pos_emb_weight):
  table_p = jnp.pad(phrase_emb_weight, ((0, 0), (0, _DP - _D)))
  return _sc_embed(phrase.astype(jnp.int32), table_p, pos_emb_weight)
